# baseline (speedup 1.0000x reference)
import jax
import jax.numpy as jnp
from jax import lax
from jax.experimental import pallas as pl
from jax.experimental.pallas import tpu as pltpu
from jax.experimental.pallas import tpu_sc as plsc

USER_N = 30000
NODE_N = 100000
HALF = 16
NP = 100352
EDGE_N = 1600000
ET = 100352
BLOCKS = 196
BLK = 512
CHUNK = 128
TILE_CH = ET // CHUNK
ROWS_PER_TILE = NP // 16
ZROWS = ROWS_PER_TILE // 8
BATCH = 4096
BPT = BATCH // 16


def _gcn_kernel(x0a, x0b, srcp, dstp, wp, uis, prod_out,
                x1a, x1b, x2a, x2b, x3a, x3b,
                src_v, dst_v, w_v, rows_v, uidx_v, iidx_v,
                acc, sem_i, sem_d, sem_g, sem_s, sem_e):
    c = lax.axis_index("c")
    s = lax.axis_index("s")

    layer_refs = [(x0a, x0b), (x1a, x1b), (x2a, x2b), (x3a, x3b)]

    for l in range(3):
        grefA, grefB = layer_refs[l]
        wrefA, wrefB = layer_refs[l + 1]

        def zfill(i, carry):
            rows_v[i, :] = jnp.zeros((HALF,), jnp.float32)
            return carry
        lax.fori_loop(0, ZROWS, zfill, 0)
        zd = []
        for q in range(ROWS_PER_TILE // ZROWS):
            zd.append(pltpu.async_copy(
                rows_v.at[pl.ds(0, ZROWS)],
                acc.at[pl.ds(s * ROWS_PER_TILE + q * ZROWS, ZROWS)], sem_e))
        for d in zd:
            d.wait()
        plsc.subcore_barrier()

        def in_descs(b, islot, dslot, fire):
            mk = pltpu.async_copy if fire else pltpu.make_async_copy
            base = s * ET + b * BLK
            return [mk(srcp.at[pl.ds(base, BLK)], src_v.at[islot],
                       sem_i.at[islot]),
                    mk(wp.at[pl.ds(base, BLK)], w_v.at[islot],
                       sem_i.at[islot]),
                    mk(dstp.at[pl.ds(base, BLK)], dst_v.at[dslot],
                       sem_d.at[dslot])]

        def g_fire(islot):
            @pl.when(c == 0)
            def _():
                pltpu.async_copy(grefA.at[src_v.at[islot]],
                                 rows_v.at[pl.ds(islot * BLK, BLK)],
                                 sem_g.at[islot])
            @pl.when(c == 1)
            def _():
                pltpu.async_copy(grefB.at[src_v.at[islot]],
                                 rows_v.at[pl.ds(islot * BLK, BLK)],
                                 sem_g.at[islot])

        def g_wait(islot):
            pltpu.make_async_copy(grefA.at[src_v.at[islot]],
                                  rows_v.at[pl.ds(islot * BLK, BLK)],
                                  sem_g.at[islot]).wait()

        def s_fire(islot, dslot):
            pltpu.async_copy(rows_v.at[pl.ds(islot * BLK, BLK)],
                             acc.at[dst_v.at[dslot]], sem_s.at[islot],
                             add=True)

        def s_wait(islot, dslot):
            pltpu.make_async_copy(rows_v.at[pl.ds(islot * BLK, BLK)],
                                  acc.at[dst_v.at[dslot]],
                                  sem_s.at[islot]).wait()

        def scale(islot):
            base = islot * BLK
            @plsc.parallel_loop(0, BLK // 16, unroll=4)
            def _scale(g):
                wv = w_v[islot, pl.ds(g * 16, 16)]
                for u in range(16):
                    e = base + g * 16 + u
                    rows_v[e, :] = rows_v[e, :] * wv[u]

        in_descs(0, 0, 0, True)
        for d in in_descs(0, 0, 0, False):
            d.wait()
        g_fire(0)
        in_descs(1, 1, 1, True)

        def block_body(b, carry):
            p = b & 1
            q = 1 - p
            d0 = lax.rem(b, 3)
            d1 = lax.rem(b + 1, 3)
            d2 = lax.rem(b + 2, 3)

            @pl.when(b >= 1)
            def _():
                s_wait(q, d2)

            @pl.when(b + 1 < BLOCKS)
            def _():
                for d in in_descs(b + 1, q, d1, False):
                    d.wait()
                g_fire(q)

            g_wait(p)
            scale(p)
            s_fire(p, d0)

            @pl.when(b + 2 < BLOCKS)
            def _():
                in_descs(b + 2, p, d2, True)
            return carry
        lax.fori_loop(0, BLOCKS, block_body, 0)
        s_wait(1, 0)

        plsc.subcore_barrier()
        sl = pl.ds(s * ROWS_PER_TILE, ROWS_PER_TILE)
        @pl.when(c == 0)
        def _():
            pltpu.sync_copy(acc.at[sl], wrefA.at[sl])
        @pl.when(c == 1)
        def _():
            pltpu.sync_copy(acc.at[sl], wrefB.at[sl])
        plsc.subcore_barrier()

    pltpu.sync_copy(uis.at[pl.ds(s * 2, 2)], uidx_v)
    pltpu.sync_copy(uis.at[pl.ds(32 + s * 2, 2)], iidx_v)
    for l in range(4):
        pA, pB = layer_refs[l]
        add = l > 0

        @pl.when(c == 0)
        def _():
            for j in range(2):
                pltpu.async_copy(pA.at[uidx_v.at[j]],
                                 rows_v.at[pl.ds(j * CHUNK, CHUNK)],
                                 sem_e, add=add)
                pltpu.async_copy(pA.at[iidx_v.at[j]],
                                 rows_v.at[pl.ds(BPT + j * CHUNK, CHUNK)],
                                 sem_e, add=add)

        @pl.when(c == 1)
        def _():
            for j in range(2):
                pltpu.async_copy(pB.at[uidx_v.at[j]],
                                 rows_v.at[pl.ds(j * CHUNK, CHUNK)],
                                 sem_e, add=add)
                pltpu.async_copy(pB.at[iidx_v.at[j]],
                                 rows_v.at[pl.ds(BPT + j * CHUNK, CHUNK)],
                                 sem_e, add=add)

        for j in range(2):
            pltpu.make_async_copy(pA.at[uidx_v.at[j]],
                                  rows_v.at[pl.ds(j * CHUNK, CHUNK)],
                                  sem_e).wait()
            pltpu.make_async_copy(pA.at[iidx_v.at[j]],
                                  rows_v.at[pl.ds(BPT + j * CHUNK, CHUNK)],
                                  sem_e).wait()

    def pbody(i, carry):
        rows_v[i, :] = rows_v[i, :] * rows_v[BPT + i, :]
        return carry
    lax.fori_loop(0, BPT, pbody, 0)
    pltpu.sync_copy(rows_v.at[pl.ds(0, BPT)],
                    prod_out.at[pl.ds(c * BATCH + s * BPT, BPT)])


@jax.jit
def kernel(user_emb, item_emb, edge_index, edge_weight, users, items):
    f32 = jnp.float32
    i32 = jnp.int32

    pad = jnp.zeros((NP - NODE_N, HALF), f32)
    x0a = jnp.concatenate([user_emb[:, :HALF], item_emb[:, :HALF], pad], axis=0)
    x0b = jnp.concatenate([user_emb[:, HALF:], item_emb[:, HALF:], pad], axis=0)

    src = edge_index[0].astype(i32)
    dst = edge_index[1].astype(i32)

    def pad_edges(a, fill):
        a2 = a.reshape(16, EDGE_N // 16)
        p = jnp.full((16, ET - EDGE_N // 16), fill, a.dtype)
        return jnp.concatenate([a2, p], axis=1).reshape(-1)

    srcp = pad_edges(src, 0)
    dstp = pad_edges(dst, 0)
    wp = pad_edges(edge_weight, f32(0))

    unodes = users.astype(i32).reshape(32, CHUNK)
    inodes = (items.astype(i32) + USER_N).reshape(32, CHUNK)
    uis = jnp.concatenate([unodes, inodes], axis=0)

    half_t = jax.ShapeDtypeStruct((NP, HALF), f32)
    kfn = pl.kernel(
        _gcn_kernel,
        out_type=(jax.ShapeDtypeStruct((2 * BATCH, HALF), f32),
                  half_t, half_t, half_t, half_t, half_t, half_t),
        mesh=plsc.VectorSubcoreMesh(core_axis_name="c", subcore_axis_name="s"),
        compiler_params=pltpu.CompilerParams(use_tc_tiling_on_sc=False),
        scratch_types=[
            pltpu.VMEM((2, BLK), i32),
            pltpu.VMEM((3, BLK), i32),
            pltpu.VMEM((2, BLK), f32),
            pltpu.VMEM((2 * BLK, HALF), f32),
            pltpu.VMEM((2, CHUNK), i32),
            pltpu.VMEM((2, CHUNK), i32),
            pltpu.VMEM_SHARED((NP, HALF), f32),
            pltpu.SemaphoreType.DMA((2,)),
            pltpu.SemaphoreType.DMA((3,)),
            pltpu.SemaphoreType.DMA((2,)),
            pltpu.SemaphoreType.DMA((2,)),
            pltpu.SemaphoreType.DMA,
        ],
    )
    prod = kfn(x0a, x0b, srcp, dstp, wp, uis)[0]
    score = prod.reshape(2, BATCH, HALF).sum(axis=(0, 2)) * f32(1.0 / 16.0)
    return jax.nn.sigmoid(score)
